# Initial kernel scaffold; baseline (speedup 1.0000x reference)
import functools

import jax
import jax.numpy as jnp
from jax import lax
from jax.experimental import pallas as pl
from jax.experimental.pallas import tpu as pltpu
from jax.experimental.pallas import tpu_sc as plsc

N = 10000
E = 160000
D = 256
H = 128
NC = 2
NS = 16
C = 80
EPT = E // NS
NCHUNK = EPT // C
RPT = N // NS


def _sc_body(xa, xb, src, dst, z_agg, z_deg, onerows,
             agg_a, agg_b, degp0, degp1,
             idx_s, idx_d, rows, ones_v, agg_sp, degp_sp, sem):
    c = lax.axis_index("c")
    s = lax.axis_index("s")
    rs = pl.ds(s * RPT, RPT)

    pltpu.sync_copy(z_agg.at[rs], agg_sp.at[rs])
    pltpu.sync_copy(z_deg.at[rs], degp_sp.at[rs])
    pltpu.sync_copy(onerows, ones_v)
    plsc.subcore_barrier()

    do_deg = ((c == 0) & (s < NS // 2)) | ((c == 1) & (s >= NS // 2))

    def process(x_half):
        def chunk(i, carry):
            base = pl.multiple_of(s * EPT + i * C, 8)
            pltpu.sync_copy(src.at[pl.ds(base, C)], idx_s)
            pltpu.sync_copy(dst.at[pl.ds(base, C)], idx_d)
            pltpu.async_copy(x_half.at[idx_s], rows, sem).wait()
            pltpu.sync_copy(rows, agg_sp.at[idx_d], add=True)

            @pl.when(do_deg)
            def _():
                pltpu.sync_copy(ones_v, degp_sp.at[idx_d], add=True)

            return carry

        lax.fori_loop(0, NCHUNK, chunk, jnp.int32(0))

    @pl.when(c == 0)
    def _():
        process(xa)

    @pl.when(c == 1)
    def _():
        process(xb)

    plsc.subcore_barrier()

    @pl.when(c == 0)
    def _():
        pltpu.sync_copy(agg_sp.at[rs], agg_a.at[rs])
        pltpu.sync_copy(degp_sp.at[rs], degp0.at[rs])

    @pl.when(c == 1)
    def _():
        pltpu.sync_copy(agg_sp.at[rs], agg_b.at[rs])
        pltpu.sync_copy(degp_sp.at[rs], degp1.at[rs])


def _sc_aggregate(xa, xb, src, dst):
    z_agg = jnp.zeros((N, H), jnp.float32)
    z_deg = jnp.zeros((N, 16), jnp.float32)
    onerows = jnp.zeros((C, 16), jnp.float32).at[:, 0].set(1.0)

    mesh = plsc.VectorSubcoreMesh(core_axis_name="c", subcore_axis_name="s")
    f = pl.kernel(
        _sc_body,
        out_type=(
            jax.ShapeDtypeStruct((N, H), jnp.float32),
            jax.ShapeDtypeStruct((N, H), jnp.float32),
            jax.ShapeDtypeStruct((N, 16), jnp.float32),
            jax.ShapeDtypeStruct((N, 16), jnp.float32),
        ),
        mesh=mesh,
        scratch_types=[
            pltpu.VMEM((C,), jnp.int32),
            pltpu.VMEM((C,), jnp.int32),
            pltpu.VMEM((C, H), jnp.float32),
            pltpu.VMEM((C, 16), jnp.float32),
            pltpu.VMEM_SHARED((N, H), jnp.float32),
            pltpu.VMEM_SHARED((N, 16), jnp.float32),
            pltpu.SemaphoreType.DMA,
        ],
        name="sage_sc_aggregate",
    )
    return f(xa, xb, src, dst, z_agg, z_deg, onerows)


R = 1000


def _tc_body(x, aa, ab, d0, d1, Wl, bl, Wr, Wa, ba, W1, b1, W2, b2, W3p, b3p,
             out):
    deg = d0[:, 0:1] + d1[:, 0:1]
    inv = 1.0 / jnp.maximum(deg, 1.0)
    mean = jnp.concatenate([aa[...] * inv, ab[...] * inv], axis=1)
    h = (jnp.dot(mean, Wl[...], preferred_element_type=jnp.float32)
         + jnp.dot(x[...], Wr[...], preferred_element_type=jnp.float32)
         + bl[...])
    h = jnp.maximum(h, 0.0)
    h = jnp.maximum(jnp.dot(h, Wa[...], preferred_element_type=jnp.float32)
                    + ba[...], 0.0)
    h = jnp.maximum(jnp.dot(h, W1[...], preferred_element_type=jnp.float32)
                    + b1[...], 0.0)
    h = jnp.maximum(jnp.dot(h, W2[...], preferred_element_type=jnp.float32)
                    + b2[...], 0.0)
    out[...] = (jnp.dot(h, W3p[...], preferred_element_type=jnp.float32)
                + b3p[...])


def _tc_dense(x, aa, ab, d0, d1, Wl, bl, Wr, Wa, ba, W1, b1, W2, b2, W3, b3):
    W3p = jnp.pad(W3, ((0, 0), (0, 125)))
    b3p = jnp.pad(b3, (0, 125))
    nblk = N // R

    def row_spec(cols):
        return pl.BlockSpec((R, cols), lambda i: (i, 0))

    def full_spec(arr):
        nd = arr.ndim
        return pl.BlockSpec(arr.shape, (lambda n: (lambda i: (0,) * n))(nd))

    weights = (Wl, bl, Wr, Wa, ba, W1, b1, W2, b2, W3p, b3p)
    grid_spec = pl.GridSpec(
        grid=(nblk,),
        in_specs=[row_spec(D), row_spec(H), row_spec(H), row_spec(16),
                  row_spec(16)] + [full_spec(w) for w in weights],
        out_specs=row_spec(H),
    )
    return pl.pallas_call(
        _tc_body,
        grid_spec=grid_spec,
        out_shape=jax.ShapeDtypeStruct((N, H), jnp.float32),
    )(x, aa, ab, d0, d1, *weights)


@jax.jit
def kernel(x, edge_index, W_l, b_l, W_r, W_a, b_a, W_1, b_1, W_2, b_2, W_3,
           b_3):
    xa = x[:, :H]
    xb = x[:, H:]
    src = edge_index[0]
    dst = edge_index[1]
    agg_a, agg_b, degp0, degp1 = _sc_aggregate(xa, xb, src, dst)
    out = _tc_dense(x, agg_a, agg_b, degp0, degp1, W_l, b_l, W_r, W_a, b_a,
                    W_1, b_1, W_2, b_2, W_3, b_3)
    return out[:, :3]


# speedup vs baseline: 4.0307x; 4.0307x over previous
import functools

import jax
import jax.numpy as jnp
from jax import lax
from jax.experimental import pallas as pl
from jax.experimental.pallas import tpu as pltpu
from jax.experimental.pallas import tpu_sc as plsc

N = 10000
E = 160000
D = 256
H = 128
NC = 2
NS = 16
C = 80
EPT = E // NS
NCHUNK = EPT // C
NRCH = N // C


def _sc_body(xcat, src2, dst, z_agg, z_deg,
             agg, degp,
             idx_s, idx_d, rows, deg_local, agg_sp, sem):
    c = lax.axis_index("c")
    s = lax.axis_index("s")

    n_i = jnp.where(s < NRCH - (NRCH // NS) * NS, NRCH // NS + 1, NRCH // NS)

    def over_row_chunks(fn):
        def body(i, carry):
            fn(pl.ds(pl.multiple_of((s + NS * i) * C, 8), C))
            return carry

        lax.fori_loop(0, n_i, body, jnp.int32(0))

    pltpu.sync_copy(z_agg, rows)
    pltpu.sync_copy(z_deg, deg_local)

    def zero_init(rs):
        pltpu.sync_copy(rows, agg_sp.at[rs])

    over_row_chunks(zero_init)
    plsc.subcore_barrier()

    ones16 = jnp.ones((16,), jnp.float32)

    def chunk(i, carry):
        base = pl.multiple_of(s * EPT + i * C, 8)
        base2 = pl.multiple_of(c * E + s * EPT + i * C, 8)
        pltpu.sync_copy(src2.at[pl.ds(base2, C)], idx_s)
        pltpu.sync_copy(dst.at[pl.ds(base, C)], idx_d)
        pltpu.async_copy(xcat.at[idx_s], rows, sem).wait()
        pltpu.sync_copy(rows, agg_sp.at[idx_d], add=True)
        for j in range(C // 16):
            plsc.addupdate_scatter(deg_local, [idx_d[pl.ds(j * 16, 16)]],
                                   ones16)
        return carry

    lax.fori_loop(0, NCHUNK, chunk, jnp.int32(0))
    plsc.subcore_barrier()

    pltpu.sync_copy(deg_local, degp.at[c, s])

    def copy_out(rs):
        pltpu.sync_copy(agg_sp.at[rs], rows)
        pltpu.sync_copy(rows, agg.at[c, rs])

    over_row_chunks(copy_out)


def _sc_aggregate(x, src, dst):
    xcat = jnp.concatenate([x[:, :H], x[:, H:]], axis=0)
    src2 = jnp.concatenate([src, src + N])
    z_agg = jnp.zeros((C, H), jnp.float32)
    z_deg = jnp.zeros((N,), jnp.float32)

    mesh = plsc.VectorSubcoreMesh(core_axis_name="c", subcore_axis_name="s")
    f = pl.kernel(
        _sc_body,
        out_type=(
            jax.ShapeDtypeStruct((NC, N, H), jnp.float32),
            jax.ShapeDtypeStruct((NC, NS, N), jnp.float32),
        ),
        mesh=mesh,
        compiler_params=pltpu.CompilerParams(needs_layout_passes=False),
        scratch_types=[
            pltpu.VMEM((C,), jnp.int32),
            pltpu.VMEM((C,), jnp.int32),
            pltpu.VMEM((C, H), jnp.float32),
            pltpu.VMEM((N,), jnp.float32),
            pltpu.VMEM_SHARED((N, H), jnp.float32),
            pltpu.SemaphoreType.DMA,
        ],
        name="sage_sc_aggregate",
    )
    return f(xcat, src2, dst, z_agg, z_deg)


R = 1000


def _tc_body(x, aa, ab, dp, Wl, bl, Wr, Wa, ba, W1, b1, W2, b2, W3p, b3p,
             out):
    deg = jnp.sum(dp[...], axis=1, keepdims=True) * 0.5
    inv = 1.0 / jnp.maximum(deg, 1.0)
    mean = jnp.concatenate([aa[...] * inv, ab[...] * inv], axis=1)
    h = (jnp.dot(mean, Wl[...], preferred_element_type=jnp.float32)
         + jnp.dot(x[...], Wr[...], preferred_element_type=jnp.float32)
         + bl[...])
    h = jnp.maximum(h, 0.0)
    h = jnp.maximum(jnp.dot(h, Wa[...], preferred_element_type=jnp.float32)
                    + ba[...], 0.0)
    h = jnp.maximum(jnp.dot(h, W1[...], preferred_element_type=jnp.float32)
                    + b1[...], 0.0)
    h = jnp.maximum(jnp.dot(h, W2[...], preferred_element_type=jnp.float32)
                    + b2[...], 0.0)
    out[...] = (jnp.dot(h, W3p[...], preferred_element_type=jnp.float32)
                + b3p[...])


def _tc_dense(x, aa, ab, degt, Wl, bl, Wr, Wa, ba, W1, b1, W2, b2, W3, b3):
    W3p = jnp.pad(W3, ((0, 0), (0, 125)))
    b3p = jnp.pad(b3, (0, 125))
    nblk = N // R

    def row_spec(cols):
        return pl.BlockSpec((R, cols), lambda i: (i, 0))

    def full_spec(arr):
        nd = arr.ndim
        return pl.BlockSpec(arr.shape, (lambda n: (lambda i: (0,) * n))(nd))

    weights = (Wl, bl, Wr, Wa, ba, W1, b1, W2, b2, W3p, b3p)
    grid_spec = pl.GridSpec(
        grid=(nblk,),
        in_specs=[row_spec(D), row_spec(H), row_spec(H),
                  row_spec(NC * NS)] + [full_spec(w) for w in weights],
        out_specs=row_spec(H),
    )
    return pl.pallas_call(
        _tc_body,
        grid_spec=grid_spec,
        out_shape=jax.ShapeDtypeStruct((N, H), jnp.float32),
    )(x, aa, ab, degt, *weights)


@jax.jit
def kernel(x, edge_index, W_l, b_l, W_r, W_a, b_a, W_1, b_1, W_2, b_2, W_3,
           b_3):
    src = edge_index[0]
    dst = edge_index[1]
    agg, degp = _sc_aggregate(x, src, dst)
    degt = degp.reshape(NC * NS, N).T
    out = _tc_dense(x, agg[0], agg[1], degt, W_l, b_l, W_r, W_a,
                    b_a, W_1, b_1, W_2, b_2, W_3, b_3)
    return out[:, :3]
